# baseline (speedup 1.0000x reference)
import functools

import jax
import jax.numpy as jnp
from jax import lax
from jax.experimental import pallas as pl
from jax.experimental.pallas import tpu as pltpu
from jax.experimental.pallas import tpu_sc as plsc

NC = 2
NS = 16
NW = NC * NS
CHUNK = 128
NBUF = 8
AHEAD = 4


@functools.partial(jax.jit, static_argnums=(2, 3))
def _sc_gather(emb, idx3, n_chunks, dim):
    mesh = plsc.VectorSubcoreMesh(core_axis_name="c", subcore_axis_name="s")

    @functools.partial(
        pl.kernel,
        mesh=mesh,
        out_type=jax.ShapeDtypeStruct((NW, n_chunks, CHUNK, dim), jnp.float32),
        scratch_types=[
            pltpu.VMEM((n_chunks, CHUNK), jnp.int32),
            pltpu.VMEM((NBUF, CHUNK, dim), jnp.float32),
            pltpu.SemaphoreType.DMA((NBUF,)),
            pltpu.SemaphoreType.DMA((NBUF,)),
        ],
        compiler_params=pltpu.CompilerParams(use_tc_tiling_on_sc=False),
    )
    def k(table_hbm, idx_hbm, out_hbm, idx_v, rows_v, gsem, ssem):
        wid = lax.axis_index("s") * NC + lax.axis_index("c")
        pltpu.sync_copy(idx_hbm.at[wid], idx_v)

        def gather_desc(g, b):
            return pltpu.make_async_copy(
                table_hbm.at[idx_v.at[g]], rows_v.at[b], gsem.at[b])

        def store_desc(g, b):
            return pltpu.make_async_copy(
                rows_v.at[b], out_hbm.at[wid, g], ssem.at[b])

        for g0 in range(AHEAD):
            gather_desc(g0, g0).start()

        def body(g, _):
            b = lax.rem(g, NBUF)
            gn = g + AHEAD
            bn = lax.rem(gn, NBUF)

            @pl.when(jnp.logical_and(gn < n_chunks, gn >= NBUF))
            def _():
                store_desc(gn - NBUF, bn).wait()

            @pl.when(gn < n_chunks)
            def _():
                gather_desc(gn, bn).start()

            gather_desc(g, b).wait()
            store_desc(g, b).start()
            return 0

        lax.fori_loop(0, n_chunks, body, 0, unroll=False)

        for j in range(NBUF):
            c = n_chunks - NBUF + j
            store_desc(c, c % NBUF).wait()

    return k(emb, idx3)


def kernel(emb, idxs):
    batch, hist = idxs.shape
    vocab, dim = emb.shape
    total = batch * hist
    assert total % (NW * CHUNK) == 0
    n_chunks = total // (NW * CHUNK)
    idx3 = idxs.astype(jnp.int32).reshape(NW, n_chunks, CHUNK)
    out = _sc_gather(emb, idx3, n_chunks, dim)
    return out.reshape(batch, hist, dim)
